# baseline (speedup 1.0000x reference)
import functools

import jax
import jax.numpy as jnp
from jax import lax
from jax.experimental import pallas as pl
from jax.experimental.pallas import tpu as pltpu
from jax.experimental.pallas import tpu_sc as plsc

B = 16384
D = 64
L = 16
NC, NS = 2, 16
NW = NC * NS
BPW = B // NW
ROWS = 2 * BPW

_mesh = plsc.VectorSubcoreMesh(
    core_axis_name="c", subcore_axis_name="s", num_cores=NC, num_subcores=NS
)


def _emb_dot_body(x_hbm, w_hbm, out_hbm, idx_v, rows_v, sums_v, out_v, sem):
    wid = lax.axis_index("s") * NC + lax.axis_index("c")
    base = wid * ROWS
    pltpu.sync_copy(x_hbm.at[pl.ds(base, ROWS)], idx_v)
    pltpu.async_copy(w_hbm.at[idx_v], rows_v, sem).wait()

    def element(i):
        acc = jnp.zeros((L,), jnp.float32)
        for k in range(D // L):
            lv = rows_v[2 * i, pl.ds(k * L, L)]
            rv = rows_v[2 * i + 1, pl.ds(k * L, L)]
            acc = acc + lv * rv
        sums_v[pl.ds(i * (L + 1), L)] = acc

    plsc.parallel_loop(0, BPW, 1, unroll=8)(element)

    lane = lax.iota(jnp.int32, L)

    def group(g):
        base = (g * L + lane) * (L + 1)
        acc = plsc.load_gather(sums_v, [base])
        for j in range(1, L):
            acc = acc + plsc.load_gather(sums_v, [base + j])
        out_v[pl.ds(g * L, L)] = 1.0 / (1.0 + jnp.exp(-acc))

    plsc.parallel_loop(0, BPW // L, 1, unroll=2)(group)
    pltpu.sync_copy(out_v, out_hbm.at[pl.ds(wid * BPW, BPW)])


_emb_dot = pl.kernel(
    _emb_dot_body,
    out_type=jax.ShapeDtypeStruct((B,), jnp.float32),
    mesh=_mesh,
    scratch_types=[
        pltpu.VMEM((ROWS,), jnp.int32),
        pltpu.VMEM((ROWS, D), jnp.float32),
        pltpu.VMEM((BPW * (L + 1),), jnp.float32),
        pltpu.VMEM((BPW,), jnp.float32),
        pltpu.SemaphoreType.DMA,
    ],
    compiler_params=pltpu.CompilerParams(
        needs_layout_passes=False, use_tc_tiling_on_sc=False
    ),
)


def kernel(x, W_g):
    return _emb_dot(x.reshape(-1), W_g)
